# baseline (speedup 1.0000x reference)
import functools
import numpy as np
import jax
import jax.numpy as jnp
from jax import lax
from jax.experimental import pallas as pl
from jax.experimental.pallas import tpu as pltpu
from jax.experimental.pallas import tpu_sc as plsc

_B, _T, _P, _D = 8, 16, 576, 768
_N = _B * _T

_WORDS = [0x8D744451, 0xB39A25C9, 0x587166EB, 0x27893CC9]
_FLAT = np.array([(w >> b) & 1 for w in _WORDS for b in range(32)], dtype=bool)
_COPIES = np.nonzero(~_FLAT)[0].tolist()
_TOKENS = np.nonzero(_FLAT)[0].tolist()

_NC, _NS = 2, 16
_NW = _NC * _NS

_S0 = _COPIES[0:32]
_S1 = _COPIES[32:64]
_S2 = [0] * _NW
_S3 = [0] * _NW
_NC3 = [1 if w < 3 else 0 for w in range(_NW)]
_t = 0
for _w in range(_NW):
    if _w < 3:
        _S2[_w] = _COPIES[64 + _w]
        _S3[_w] = _TOKENS[_t]
        _t += 1
    else:
        _S2[_w] = _TOKENS[_t]
        _S3[_w] = _TOKENS[_t + 1]
        _t += 2
assert _t == len(_TOKENS)

_CH = 48
_NCH = _P // _CH
_TR = 16
_NTW = _P // _TR


def _lookup32(table, idx):
    cur = [jnp.int32(int(v)) for v in table]
    bit = 0
    while len(cur) > 1:
        b = (idx >> bit) & 1
        cur = [jnp.where(b == 0, cur[i], cur[i + 1]) for i in range(0, len(cur), 2)]
        bit += 1
    return cur[0]


def _sc_body(x_hbm, tok_hbm, out_hbm, tokbuf, b0, b1, b2, sem_t, sem_g, sem_s, sem_w):
    wid = lax.axis_index("s") * _NC + lax.axis_index("c")
    f0 = _lookup32(_S0, wid)
    f1 = _lookup32(_S1, wid)
    f2 = _lookup32(_S2, wid)
    f3 = _lookup32(_S3, wid)
    three = _lookup32(_NC3, wid)

    tf = [
        pltpu.make_async_copy(tok_hbm, tokbuf.at[pl.ds(r, 1)], sem_t)
        for r in range(_TR)
    ]
    for c in tf:
        c.start()
    for c in tf:
        c.wait()

    def fire_tok(f):
        for j in range(_NTW):
            pltpu.make_async_copy(
                tokbuf, out_hbm.at[f, pl.ds(j * _TR, _TR)], sem_w
            ).start()

    fire_tok(f3)

    @pl.when(three == 0)
    def _():
        fire_tok(f2)

    bufs = [b0, b1, b2]

    def copy_frame(f):
        pltpu.make_async_copy(x_hbm.at[f], out_hbm.at[f], sem_g).start()

    def drain_frame(f):
        pltpu.make_async_copy(x_hbm.at[f], out_hbm.at[f], sem_g).wait()

    copy_frame(f0)
    copy_frame(f1)

    @pl.when(three != 0)
    def _():
        copy_frame(f2)

    drain_frame(f0)
    drain_frame(f1)

    @pl.when(three != 0)
    def _():
        drain_frame(f2)

    for j in range(_NTW):
        pltpu.make_async_copy(tokbuf, out_hbm.at[f3, pl.ds(j * _TR, _TR)], sem_w).wait()

    @pl.when(three == 0)
    def _():
        for j in range(_NTW):
            pltpu.make_async_copy(
                tokbuf, out_hbm.at[f2, pl.ds(j * _TR, _TR)], sem_w
            ).wait()


def kernel(x, mask_token):
    x3 = x.reshape(_N, _P, _D)
    tok = mask_token.reshape(1, _D)
    mesh = plsc.VectorSubcoreMesh(core_axis_name="c", subcore_axis_name="s")
    k = functools.partial(
        pl.kernel,
        mesh=mesh,
        out_type=jax.ShapeDtypeStruct((_N, _P, _D), jnp.float32),
        scratch_types=[
            pltpu.VMEM((_TR, _D), jnp.float32),
            pltpu.VMEM((_CH, _D), jnp.float32),
            pltpu.VMEM((_CH, _D), jnp.float32),
            pltpu.VMEM((_CH, _D), jnp.float32),
            pltpu.SemaphoreType.DMA,
            pltpu.SemaphoreType.DMA,
            pltpu.SemaphoreType.DMA,
            pltpu.SemaphoreType.DMA,
        ],
    )(_sc_body)
    out3 = k(x3, tok)
    return out3.reshape(_B, _T, _P, _D)


# speedup vs baseline: 32.3632x; 32.3632x over previous
import numpy as np
import jax
import jax.numpy as jnp
from jax.experimental import pallas as pl
from jax.experimental.pallas import tpu as pltpu

_B, _T, _P, _D = 8, 16, 576, 768
_N = _B * _T

_WORDS = [0x8D744451, 0xB39A25C9, 0x587166EB, 0x27893CC9]
_FLAT = np.array([(w >> b) & 1 for w in _WORDS for b in range(32)], dtype=bool)
_MASKED = np.nonzero(_FLAT)[0]
_UNMASKED = np.nonzero(~_FLAT)[0]
_NCP = len(_UNMASKED)
_NTOK = len(_MASKED)

_K = 10
_A = 6


def _body(x_ref, tok_ref, out_ref, tokf, ring, sem_g, sem_s, sem_t):
    tokf[...] = jnp.broadcast_to(tok_ref[0, :], (_P, _D))

    g = [
        pltpu.make_async_copy(x_ref.at[int(f)], ring.at[c % _K], sem_g)
        for c, f in enumerate(_UNMASKED)
    ]
    s = [
        pltpu.make_async_copy(ring.at[c % _K], out_ref.at[int(f)], sem_s)
        for c, f in enumerate(_UNMASKED)
    ]
    t = [pltpu.make_async_copy(tokf, out_ref.at[int(f)], sem_t) for f in _MASKED]

    for c in range(_A):
        g[c].start()
    waited_s = -1
    for c in range(_NCP):
        if c < _NTOK:
            t[c].start()
        g[c].wait()
        s[c].start()
        if c + _A < _NCP:
            if c + _A - _K >= 0:
                s[c + _A - _K].wait()
                waited_s = c + _A - _K
            g[c + _A].start()
    for c in range(waited_s + 1, _NCP):
        s[c].wait()
    for c in range(_NCP, _NTOK):
        t[c].start()
    for c in range(_NTOK):
        t[c].wait()


def kernel(x, mask_token):
    x3 = x.reshape(_N, _P, _D)
    tok = mask_token.reshape(1, _D)
    out3 = pl.pallas_call(
        _body,
        in_specs=[
            pl.BlockSpec(memory_space=pl.ANY),
            pl.BlockSpec(memory_space=pltpu.VMEM),
        ],
        out_specs=pl.BlockSpec(memory_space=pl.ANY),
        out_shape=jax.ShapeDtypeStruct((_N, _P, _D), x.dtype),
        scratch_shapes=[
            pltpu.VMEM((_P, _D), jnp.float32),
            pltpu.VMEM((_K, _P, _D), jnp.float32),
            pltpu.SemaphoreType.DMA,
            pltpu.SemaphoreType.DMA,
            pltpu.SemaphoreType.DMA,
        ],
    )(x3, tok)
    return out3.reshape(_B, _T, _P, _D)
